# baseline (speedup 1.0000x reference)
import jax
import jax.numpy as jnp
from jax.experimental import pallas as pl
from jax.experimental.pallas import tpu as pltpu


def _body(x_ref, tab_ref, o_ref):
    tab = tab_ref[...]
    V, D = tab.shape
    S = x_ref.shape[0]
    if S > V:
        fill = jnp.full((S - V, D), jnp.nan, dtype=tab.dtype)
        ext = jnp.concatenate([tab, fill], axis=0)
    else:
        ext = tab[:S]
    y = x_ref[...] + ext
    o_ref[...] = jnp.broadcast_to(y[None], o_ref.shape)


def kernel(x, emb_table):
    B, S = x.shape
    V, D = emb_table.shape
    BB = 8
    return pl.pallas_call(
        _body,
        grid=(B // BB,),
        in_specs=[
            pl.BlockSpec((S, D), lambda i: (0, 0)),
            pl.BlockSpec((V, D), lambda i: (0, 0)),
        ],
        out_specs=pl.BlockSpec((BB, S, D), lambda i: (i, 0, 0)),
        out_shape=jax.ShapeDtypeStruct((B, S, D), x.dtype),
        compiler_params=pltpu.CompilerParams(
            dimension_semantics=("arbitrary",)),
    )(x, emb_table)


# speedup vs baseline: 1.1718x; 1.1718x over previous
import jax
import jax.numpy as jnp
from jax.experimental import pallas as pl
from jax.experimental.pallas import tpu as pltpu


def _body(x_ref, tab_ref, o_ref):
    tab = tab_ref[...]
    V, D = tab.shape
    S = x_ref.shape[0]
    if S > V:
        fill = jnp.full((S - V, D), jnp.nan, dtype=tab.dtype)
        ext = jnp.concatenate([tab, fill], axis=0)
    else:
        ext = tab[:S]
    y = x_ref[...] + ext
    o_ref[...] = jnp.broadcast_to(y[None], o_ref.shape)


def kernel(x, emb_table):
    B, S = x.shape
    V, D = emb_table.shape
    BB = 16
    return pl.pallas_call(
        _body,
        grid=(B // BB,),
        in_specs=[
            pl.BlockSpec((S, D), lambda i: (0, 0)),
            pl.BlockSpec((V, D), lambda i: (0, 0)),
        ],
        out_specs=pl.BlockSpec((BB, S, D), lambda i: (i, 0, 0)),
        out_shape=jax.ShapeDtypeStruct((B, S, D), x.dtype),
        compiler_params=pltpu.CompilerParams(
            dimension_semantics=("parallel",)),
    )(x, emb_table)
